# baseline (speedup 1.0000x reference)
import functools

import jax
import jax.numpy as jnp
from jax import lax
from jax.experimental import pallas as pl
from jax.experimental.pallas import tpu as pltpu
from jax.experimental.pallas import tpu_sc as plsc

N = 10000
E = 160000
W = 32
K2 = 128
DEPTH = 2

_SL = 128
_NS = 16
_NC = 2
_NW = _NS * _NC
_NSTREAM = E // _SL
_HALF = E // _NC
_HSTREAM = _HALF // _SL
_ROWS_T = N // _NS

_mesh = functools.partial(
    plsc.VectorSubcoreMesh, core_axis_name="c", subcore_axis_name="s")

_sc_params = pltpu.CompilerParams(use_tc_tiling_on_sc=False)


def _sc_gather_body(h_hbm, src_hbm, out_hbm, idx_v, rows_v, sem):
    c = lax.axis_index("c")
    s = lax.axis_index("s")
    w = s * _NC + c

    def body(i, carry):
        t = i * _NW + w

        @pl.when(t < _NSTREAM)
        def _():
            off = t * _SL
            pltpu.sync_copy(src_hbm.at[pl.ds(off, _SL)], idx_v)
            pltpu.async_copy(h_hbm.at[idx_v], rows_v, sem).wait()
            pltpu.sync_copy(rows_v, out_hbm.at[pl.ds(off, _SL)])

        return carry

    lax.fori_loop(0, (_NSTREAM + _NW - 1) // _NW, body, 0)


def _sc_gather(h, src):
    return pl.kernel(
        _sc_gather_body,
        out_type=jax.ShapeDtypeStruct((E, W), jnp.float32),
        mesh=_mesh(),
        scratch_types=[
            pltpu.VMEM((_SL,), jnp.int32),
            pltpu.VMEM((_SL, W), jnp.float32),
            pltpu.SemaphoreType.DMA,
        ],
        compiler_params=_sc_params,
    )(h, src)


def _sc_scatter_body(msg_hbm, dst_hbm, zero_hbm, out_hbm, idx_v, rows_v, acc_sh, sem):
    c = lax.axis_index("c")
    s = lax.axis_index("s")
    r0 = s * _ROWS_T
    pltpu.sync_copy(zero_hbm.at[pl.ds(r0, _ROWS_T)], acc_sh.at[pl.ds(r0, _ROWS_T)])
    plsc.subcore_barrier()

    def body(i, carry):
        t = i * _NS + s

        @pl.when(t < _HSTREAM)
        def _():
            off = c * _HALF + t * _SL
            pltpu.sync_copy(dst_hbm.at[pl.ds(off, _SL)], idx_v)
            pltpu.sync_copy(msg_hbm.at[pl.ds(off, _SL)], rows_v)
            pltpu.sync_copy(rows_v, acc_sh.at[idx_v], add=True)

        return carry

    lax.fori_loop(0, (_HSTREAM + _NS - 1) // _NS, body, 0)
    plsc.subcore_barrier()
    pltpu.sync_copy(acc_sh.at[pl.ds(r0, _ROWS_T)], out_hbm.at[c, pl.ds(r0, _ROWS_T)])


def _sc_scatter(msg, dst, zeros):
    return pl.kernel(
        _sc_scatter_body,
        out_type=jax.ShapeDtypeStruct((_NC, N, W), jnp.float32),
        mesh=_mesh(),
        scratch_types=[
            pltpu.VMEM((_SL,), jnp.int32),
            pltpu.VMEM((_SL, W), jnp.float32),
            pltpu.VMEM_SHARED((N, W), jnp.float32),
            pltpu.SemaphoreType.DMA,
        ],
        compiler_params=_sc_params,
    )(msg, dst, zeros)


def _sc_count_body(dst_hbm, zero_hbm, out_hbm, idx_v, ones_v, acc_sh, sem):
    c = lax.axis_index("c")
    s = lax.axis_index("s")

    def fill(i, carry):
        ones_v[i, :] = jnp.ones((16,), jnp.float32)
        return carry

    lax.fori_loop(0, _SL, fill, 0)
    r0 = s * _ROWS_T
    pltpu.sync_copy(zero_hbm.at[pl.ds(r0, _ROWS_T)], acc_sh.at[pl.ds(r0, _ROWS_T)])
    plsc.subcore_barrier()

    def body(i, carry):
        t = i * _NS + s

        @pl.when(t < _HSTREAM)
        def _():
            off = c * _HALF + t * _SL
            pltpu.sync_copy(dst_hbm.at[pl.ds(off, _SL)], idx_v)
            pltpu.sync_copy(ones_v, acc_sh.at[idx_v], add=True)

        return carry

    lax.fori_loop(0, (_HSTREAM + _NS - 1) // _NS, body, 0)
    plsc.subcore_barrier()
    pltpu.sync_copy(acc_sh.at[pl.ds(r0, _ROWS_T)], out_hbm.at[c, pl.ds(r0, _ROWS_T)])


def _sc_count(dst, zeros16):
    return pl.kernel(
        _sc_count_body,
        out_type=jax.ShapeDtypeStruct((_NC, N, 16), jnp.float32),
        mesh=_mesh(),
        scratch_types=[
            pltpu.VMEM((_SL,), jnp.int32),
            pltpu.VMEM((_SL, 16), jnp.float32),
            pltpu.VMEM_SHARED((N, 16), jnp.float32),
            pltpu.SemaphoreType.DMA,
        ],
        compiler_params=_sc_params,
    )(dst, zeros16)


def _layer_norm(h, g, b):
    mu = jnp.mean(h, axis=-1, keepdims=True)
    var = jnp.mean((h - mu) ** 2, axis=-1, keepdims=True)
    return (h - mu) * lax.rsqrt(var + 1e-5) * g + b


def _node_body(x_ref, w1_ref, b1_ref, w2_ref, b2_ref, g_ref, bt_ref, out_ref):
    f32 = jnp.float32
    h = jnp.maximum(jnp.dot(x_ref[...], w1_ref[...], preferred_element_type=f32)
                    + b1_ref[...], 0.0)
    h = jnp.dot(h, w2_ref[...], preferred_element_type=f32) + b2_ref[...]
    out_ref[...] = _layer_norm(h, g_ref[...], bt_ref[...])


def _tc_node_encode(x, w1, b1, w2, b2, g, bt):
    return pl.pallas_call(
        _node_body,
        out_shape=jax.ShapeDtypeStruct((N, W), jnp.float32),
    )(x, w1, b1.reshape(1, W), w2, b2.reshape(1, W), g.reshape(1, W),
      bt.reshape(1, W))


_B = 256


def _msg_body(ea_ref, hs_ref, eW1_ref, eb1_ref, eW2_ref, eb2_ref, eg_ref,
              ebt_ref, kW1_ref, kb1_ref, kW2_ref, kb2_ref, kW3_ref, kb3_ref,
              out_ref):
    f32 = jnp.float32
    e = jnp.maximum(jnp.dot(ea_ref[...], eW1_ref[...], preferred_element_type=f32)
                    + eb1_ref[...], 0.0)
    e = jnp.dot(e, eW2_ref[...], preferred_element_type=f32) + eb2_ref[...]
    e = _layer_norm(e, eg_ref[...], ebt_ref[...])
    k1 = jnp.maximum(jnp.dot(e, kW1_ref[...], preferred_element_type=f32)
                     + kb1_ref[...], 0.0)
    k2 = jnp.maximum(jnp.dot(k1, kW2_ref[...], preferred_element_type=f32)
                     + kb2_ref[...], 0.0)
    wk = jnp.dot(k2, kW3_ref[...], preferred_element_type=f32) + kb3_ref[...]
    hs = hs_ref[...]
    msg = wk[:, 0:W] * hs[:, 0:1]
    for j in range(1, W):
        msg = msg + wk[:, j * W:(j + 1) * W] * hs[:, j:j + 1]
    out_ref[...] = msg


def _tc_message(edge_attr, hs, eW1, eb1, eW2, eb2, eg, ebt, kW1l, kb1l, kW2l,
                kb2l, kW3l, kb3l):
    def cmap(i):
        return (i, 0)

    def zmap(i):
        return (0, 0)

    return pl.pallas_call(
        _msg_body,
        grid=(E // _B,),
        in_specs=[
            pl.BlockSpec((_B, 4), cmap),
            pl.BlockSpec((_B, W), cmap),
            pl.BlockSpec((4, W), zmap),
            pl.BlockSpec((1, W), zmap),
            pl.BlockSpec((W, W), zmap),
            pl.BlockSpec((1, W), zmap),
            pl.BlockSpec((1, W), zmap),
            pl.BlockSpec((1, W), zmap),
            pl.BlockSpec((W, K2), zmap),
            pl.BlockSpec((1, K2), zmap),
            pl.BlockSpec((K2, K2), zmap),
            pl.BlockSpec((1, K2), zmap),
            pl.BlockSpec((K2, W * W), zmap),
            pl.BlockSpec((1, W * W), zmap),
        ],
        out_specs=pl.BlockSpec((_B, W), cmap),
        out_shape=jax.ShapeDtypeStruct((E, W), jnp.float32),
        compiler_params=pltpu.CompilerParams(
            dimension_semantics=("arbitrary",)),
    )(edge_attr, hs, eW1, eb1.reshape(1, W), eW2, eb2.reshape(1, W),
      eg.reshape(1, W), ebt.reshape(1, W), kW1l, kb1l.reshape(1, K2), kW2l,
      kb2l.reshape(1, K2), kW3l, kb3l.reshape(1, W * W))


def _agg(s_ref, cnt_ref):
    cnt = cnt_ref[0, :, 0:1] + cnt_ref[1, :, 0:1]
    return (s_ref[0, :, :] + s_ref[1, :, :]) / jnp.maximum(cnt, 1.0)


def _update_body(h_ref, s_ref, cnt_ref, out_ref):
    out_ref[...] = jnp.maximum(h_ref[...] + _agg(s_ref, cnt_ref), 0.0)


def _tc_update(h, s, cnt):
    return pl.pallas_call(
        _update_body,
        out_shape=jax.ShapeDtypeStruct((N, W), jnp.float32),
    )(h, s, cnt)


def _update_decode_body(h_ref, s_ref, cnt_ref, w1_ref, b1_ref, w2_ref, b2_ref,
                        out_ref):
    f32 = jnp.float32
    h2 = jnp.maximum(h_ref[...] + _agg(s_ref, cnt_ref), 0.0)
    t = jnp.maximum(jnp.dot(h2, w1_ref[...], preferred_element_type=f32)
                    + b1_ref[...], 0.0)
    out_ref[...] = jnp.dot(t, w2_ref[...], preferred_element_type=f32) + b2_ref[...]


def _tc_update_decode(h, s, cnt, w1, b1, w2, b2):
    return pl.pallas_call(
        _update_decode_body,
        out_shape=jax.ShapeDtypeStruct((N, 1), jnp.float32),
    )(h, s, cnt, w1, b1.reshape(1, W), w2, b2.reshape(1, 1))


def kernel(x, edge_index, edge_attr, nW1, nb1, nW2, nb2, ng, nbt, eW1, eb1,
           eW2, eb2, eg, ebt, kW1, kb1, kW2, kb2, kW3, kb3, dW1, db1, dW2,
           db2):
    src = edge_index[0]
    dst = edge_index[1]
    kW3p = kW3.reshape(DEPTH, K2, W, W).transpose(0, 1, 3, 2).reshape(
        DEPTH, K2, W * W)
    kb3p = kb3.reshape(DEPTH, W, W).transpose(0, 2, 1).reshape(DEPTH, W * W)
    z32 = jnp.zeros((N, W), jnp.float32)
    z16 = jnp.zeros((N, 16), jnp.float32)

    h = _tc_node_encode(x, nW1, nb1, nW2, nb2, ng, nbt)
    cnt = _sc_count(dst, z16)
    out = None
    for l in range(DEPTH):
        hs = _sc_gather(h, src)
        msg = _tc_message(edge_attr, hs, eW1, eb1, eW2, eb2, eg, ebt,
                          kW1[l], kb1[l], kW2[l], kb2[l], kW3p[l], kb3p[l])
        s = _sc_scatter(msg, dst, z32)
        if l < DEPTH - 1:
            h = _tc_update(h, s, cnt)
        else:
            out = _tc_update_decode(h, s, cnt, dW1, db1, dW2, db2)
    return out


# speedup vs baseline: 3.1391x; 3.1391x over previous
import functools

import jax
import jax.numpy as jnp
from jax import lax
from jax.experimental import pallas as pl
from jax.experimental.pallas import tpu as pltpu
from jax.experimental.pallas import tpu_sc as plsc

N = 10000
E = 160000
W = 32
K2 = 128
DEPTH = 2

_SL = 128
_NS = 16
_NC = 2
_NW = _NS * _NC
_NSTREAM = E // _SL
_HALF = E // _NC
_HSTREAM = _HALF // _SL
_ROWS_T = N // _NS

_mesh = functools.partial(
    plsc.VectorSubcoreMesh, core_axis_name="c", subcore_axis_name="s")

_sc_params = pltpu.CompilerParams(use_tc_tiling_on_sc=False)


def _sc_gather_body(h_hbm, src_hbm, out_hbm, idx_v, rows_v, sem):
    c = lax.axis_index("c")
    s = lax.axis_index("s")
    w = s * _NC + c

    def body(i, carry):
        t = i * _NW + w

        @pl.when(t < _NSTREAM)
        def _():
            off = t * _SL
            pltpu.sync_copy(src_hbm.at[pl.ds(off, _SL)], idx_v)
            pltpu.async_copy(h_hbm.at[idx_v], rows_v, sem).wait()
            pltpu.sync_copy(rows_v, out_hbm.at[pl.ds(off, _SL)])

        return carry

    lax.fori_loop(0, (_NSTREAM + _NW - 1) // _NW, body, 0)


def _sc_gather(h, src):
    return pl.kernel(
        _sc_gather_body,
        out_type=jax.ShapeDtypeStruct((E, W), jnp.float32),
        mesh=_mesh(),
        scratch_types=[
            pltpu.VMEM((_SL,), jnp.int32),
            pltpu.VMEM((_SL, W), jnp.float32),
            pltpu.SemaphoreType.DMA,
        ],
        compiler_params=_sc_params,
    )(h, src)


def _sc_scatter_body(msg_hbm, dst_hbm, zero_hbm, out_hbm, idx_v, rows_v, acc_sh, sem):
    c = lax.axis_index("c")
    s = lax.axis_index("s")
    r0 = s * _ROWS_T
    pltpu.sync_copy(zero_hbm.at[pl.ds(r0, _ROWS_T)], acc_sh.at[pl.ds(r0, _ROWS_T)])
    plsc.subcore_barrier()

    def body(i, carry):
        t = i * _NS + s

        @pl.when(t < _HSTREAM)
        def _():
            off = c * _HALF + t * _SL
            pltpu.sync_copy(dst_hbm.at[pl.ds(off, _SL)], idx_v)
            pltpu.sync_copy(msg_hbm.at[pl.ds(off, _SL)], rows_v)
            pltpu.sync_copy(rows_v, acc_sh.at[idx_v], add=True)

        return carry

    lax.fori_loop(0, (_HSTREAM + _NS - 1) // _NS, body, 0)
    plsc.subcore_barrier()
    pltpu.sync_copy(acc_sh.at[pl.ds(r0, _ROWS_T)], out_hbm.at[c, pl.ds(r0, _ROWS_T)])


def _sc_scatter(msg, dst, zeros):
    return pl.kernel(
        _sc_scatter_body,
        out_type=jax.ShapeDtypeStruct((_NC, N, W), jnp.float32),
        mesh=_mesh(),
        scratch_types=[
            pltpu.VMEM((_SL,), jnp.int32),
            pltpu.VMEM((_SL, W), jnp.float32),
            pltpu.VMEM_SHARED((N, W), jnp.float32),
            pltpu.SemaphoreType.DMA,
        ],
        compiler_params=_sc_params,
    )(msg, dst, zeros)


def _sc_count_body(dst_hbm, zero_hbm, out_hbm, idx_v, ones_v, acc_sh, sem):
    c = lax.axis_index("c")
    s = lax.axis_index("s")

    def fill(i, carry):
        ones_v[i, :] = jnp.ones((16,), jnp.float32)
        return carry

    lax.fori_loop(0, _SL, fill, 0)
    r0 = s * _ROWS_T
    pltpu.sync_copy(zero_hbm.at[pl.ds(r0, _ROWS_T)], acc_sh.at[pl.ds(r0, _ROWS_T)])
    plsc.subcore_barrier()

    def body(i, carry):
        t = i * _NS + s

        @pl.when(t < _HSTREAM)
        def _():
            off = c * _HALF + t * _SL
            pltpu.sync_copy(dst_hbm.at[pl.ds(off, _SL)], idx_v)
            pltpu.sync_copy(ones_v, acc_sh.at[idx_v], add=True)

        return carry

    lax.fori_loop(0, (_HSTREAM + _NS - 1) // _NS, body, 0)
    plsc.subcore_barrier()
    pltpu.sync_copy(acc_sh.at[pl.ds(r0, _ROWS_T)], out_hbm.at[c, pl.ds(r0, _ROWS_T)])


def _sc_count(dst, zeros16):
    return pl.kernel(
        _sc_count_body,
        out_type=jax.ShapeDtypeStruct((_NC, N, 16), jnp.float32),
        mesh=_mesh(),
        scratch_types=[
            pltpu.VMEM((_SL,), jnp.int32),
            pltpu.VMEM((_SL, 16), jnp.float32),
            pltpu.VMEM_SHARED((N, 16), jnp.float32),
            pltpu.SemaphoreType.DMA,
        ],
        compiler_params=_sc_params,
    )(dst, zeros16)


def _layer_norm_c(hc, m, g, b):
    var = jnp.dot(hc * hc, m, preferred_element_type=jnp.float32)
    return hc * lax.rsqrt(var + 1e-5) * g + b


def _node_body(x_ref, w1_ref, b1_ref, w2_ref, b2_ref, m_ref, g_ref, bt_ref,
               out_ref):
    f32 = jnp.float32
    h = jnp.maximum(jnp.dot(x_ref[...], w1_ref[...], preferred_element_type=f32)
                    + b1_ref[...], 0.0)
    h = jnp.dot(h, w2_ref[...], preferred_element_type=f32) + b2_ref[...]
    out_ref[...] = _layer_norm_c(h, m_ref[...], g_ref[...], bt_ref[...])


def _tc_node_encode(x, w1, b1, w2c, b2c, m, g, bt):
    return pl.pallas_call(
        _node_body,
        out_shape=jax.ShapeDtypeStruct((N, W), jnp.float32),
    )(x, w1, b1.reshape(1, W), w2c, b2c.reshape(1, W), m, g.reshape(1, W),
      bt.reshape(1, W))


_B = 640


def _msg_body(ea_ref, hs_ref, eW1_ref, eb1_ref, eW2_ref, eb2_ref, m_ref,
              eg_ref, ebt_ref, kW1_ref, kb1_ref, kW2_ref, kb2_ref, kW3_ref,
              kb3_ref, s_ref, out_ref):
    f32 = jnp.float32
    e = jnp.maximum(jnp.dot(ea_ref[...], eW1_ref[...], preferred_element_type=f32)
                    + eb1_ref[...], 0.0)
    e = jnp.dot(e, eW2_ref[...], preferred_element_type=f32) + eb2_ref[...]
    e = _layer_norm_c(e, m_ref[...], eg_ref[...], ebt_ref[...])
    k1 = jnp.maximum(jnp.dot(e, kW1_ref[...], preferred_element_type=f32)
                     + kb1_ref[...], 0.0)
    k2 = jnp.maximum(jnp.dot(k1.astype(jnp.bfloat16), kW2_ref[...],
                             preferred_element_type=f32)
                     + kb2_ref[...], 0.0)
    wk = jnp.dot(k2.astype(jnp.bfloat16), kW3_ref[...],
                 preferred_element_type=f32) + kb3_ref[...]
    hst = jnp.concatenate([hs_ref[...]] * W, axis=-1)
    prod = wk * hst
    out_ref[...] = jnp.dot(prod, s_ref[...], preferred_element_type=f32)


def _tc_message(edge_attr, hs, eW1, eb1, eW2c, eb2c, m, eg, ebt, kW1l, kb1l,
                kW2l, kb2l, kW3l, kb3l, s):
    def cmap(i):
        return (i, 0)

    def zmap(i):
        return (0, 0)

    return pl.pallas_call(
        _msg_body,
        grid=(E // _B,),
        in_specs=[
            pl.BlockSpec((_B, 4), cmap),
            pl.BlockSpec((_B, W), cmap),
            pl.BlockSpec((4, W), zmap),
            pl.BlockSpec((1, W), zmap),
            pl.BlockSpec((W, W), zmap),
            pl.BlockSpec((1, W), zmap),
            pl.BlockSpec((W, W), zmap),
            pl.BlockSpec((1, W), zmap),
            pl.BlockSpec((1, W), zmap),
            pl.BlockSpec((W, K2), zmap),
            pl.BlockSpec((1, K2), zmap),
            pl.BlockSpec((K2, K2), zmap),
            pl.BlockSpec((1, K2), zmap),
            pl.BlockSpec((K2, W * W), zmap),
            pl.BlockSpec((1, W * W), zmap),
            pl.BlockSpec((W * W, W), zmap),
        ],
        out_specs=pl.BlockSpec((_B, W), cmap),
        out_shape=jax.ShapeDtypeStruct((E, W), jnp.float32),
        compiler_params=pltpu.CompilerParams(
            dimension_semantics=("arbitrary",)),
    )(edge_attr, hs, eW1, eb1.reshape(1, W), eW2c, eb2c.reshape(1, W), m,
      eg.reshape(1, W), ebt.reshape(1, W), kW1l, kb1l.reshape(1, K2), kW2l,
      kb2l.reshape(1, K2), kW3l, kb3l.reshape(1, W * W), s)


def _agg(s_ref, cnt_ref):
    cnt = cnt_ref[0, :, 0:1] + cnt_ref[1, :, 0:1]
    return (s_ref[0, :, :] + s_ref[1, :, :]) / jnp.maximum(cnt, 1.0)


def _update_body(h_ref, s_ref, cnt_ref, out_ref):
    out_ref[...] = jnp.maximum(h_ref[...] + _agg(s_ref, cnt_ref), 0.0)


def _tc_update(h, s, cnt):
    return pl.pallas_call(
        _update_body,
        out_shape=jax.ShapeDtypeStruct((N, W), jnp.float32),
    )(h, s, cnt)


def _update_decode_body(h_ref, s_ref, cnt_ref, w1_ref, b1_ref, w2_ref, b2_ref,
                        out_ref):
    f32 = jnp.float32
    h2 = jnp.maximum(h_ref[...] + _agg(s_ref, cnt_ref), 0.0)
    t = jnp.maximum(jnp.dot(h2, w1_ref[...], preferred_element_type=f32)
                    + b1_ref[...], 0.0)
    out_ref[...] = jnp.dot(t, w2_ref[...], preferred_element_type=f32) + b2_ref[...]


def _tc_update_decode(h, s, cnt, w1, b1, w2, b2):
    return pl.pallas_call(
        _update_decode_body,
        out_shape=jax.ShapeDtypeStruct((N, 1), jnp.float32),
    )(h, s, cnt, w1, b1.reshape(1, W), w2, b2.reshape(1, 1))


def kernel(x, edge_index, edge_attr, nW1, nb1, nW2, nb2, ng, nbt, eW1, eb1,
           eW2, eb2, eg, ebt, kW1, kb1, kW2, kb2, kW3, kb3, dW1, db1, dW2,
           db2):
    src = edge_index[0]
    dst = edge_index[1]
    z32 = jnp.zeros((N, W), jnp.float32)
    z16 = jnp.zeros((N, 16), jnp.float32)
    cen = jnp.eye(W, dtype=jnp.float32) - 1.0 / W
    m = jnp.full((W, W), 1.0 / W, jnp.float32)
    smat = jnp.kron(jnp.eye(W, dtype=jnp.float32), jnp.ones((W, 1), jnp.float32))
    nW2c = nW2 @ cen
    nb2c = nb2 @ cen
    eW2c = eW2 @ cen
    eb2c = eb2 @ cen

    h = _tc_node_encode(x, nW1, nb1, nW2c, nb2c, m, ng, nbt)
    cnt = _sc_count(dst, z16)
    out = None
    for l in range(DEPTH):
        hs = _sc_gather(h, src)
        msg = _tc_message(edge_attr, hs, eW1, eb1, eW2c, eb2c, m, eg, ebt,
                          kW1[l], kb1[l], kW2[l].astype(jnp.bfloat16),
                          kb2[l], kW3[l].astype(jnp.bfloat16), kb3[l], smat)
        s = _sc_scatter(msg, dst, z32)
        if l < DEPTH - 1:
            h = _tc_update(h, s, cnt)
        else:
            out = _tc_update_decode(h, s, cnt, dW1, db1, dW2, db2)
    return out


# speedup vs baseline: 3.4972x; 1.1141x over previous
import functools

import jax
import jax.numpy as jnp
from jax import lax
from jax.experimental import pallas as pl
from jax.experimental.pallas import tpu as pltpu
from jax.experimental.pallas import tpu_sc as plsc

N = 10000
E = 160000
W = 32
K2 = 128
DEPTH = 2

_SL = 128
_NS = 16
_NC = 2
_NW = _NS * _NC
_NSTREAM = E // _SL
_HALF = E // _NC
_HSTREAM = _HALF // _SL
_ROWS_T = N // _NS

_mesh = functools.partial(
    plsc.VectorSubcoreMesh, core_axis_name="c", subcore_axis_name="s")

_sc_params = pltpu.CompilerParams(use_tc_tiling_on_sc=False)


_GG = 8
_NGRP = (_NSTREAM + _NW * _GG - 1) // (_NW * _GG)


def _sc_gather_body(h_hbm, src_hbm, out_hbm, idx_v, rows_v, sem_i, sem_g,
                    sem_o):
    c = lax.axis_index("c")
    s = lax.axis_index("s")
    w = s * _NC + c

    def group(g, carry):
        b = g % 2

        for k in range(_GG):
            t = ((g - 2) * _GG + k) * _NW + w

            @pl.when((g >= 2) & (t < _NSTREAM))
            def _(t=t, k=k):
                pltpu.make_async_copy(
                    rows_v.at[b, pl.ds(k * _SL, _SL)],
                    out_hbm.at[pl.ds(t * _SL, _SL)], sem_o).wait()

        for k in range(_GG):
            t = (g * _GG + k) * _NW + w

            @pl.when(t < _NSTREAM)
            def _(t=t, k=k):
                pltpu.async_copy(src_hbm.at[pl.ds(t * _SL, _SL)],
                                 idx_v.at[k], sem_i)

        for k in range(_GG):
            t = (g * _GG + k) * _NW + w

            @pl.when(t < _NSTREAM)
            def _(t=t, k=k):
                pltpu.make_async_copy(src_hbm.at[pl.ds(t * _SL, _SL)],
                                      idx_v.at[k], sem_i).wait()
                pltpu.async_copy(h_hbm.at[idx_v.at[k]],
                                 rows_v.at[b, pl.ds(k * _SL, _SL)], sem_g)

        for k in range(_GG):
            t = (g * _GG + k) * _NW + w

            @pl.when(t < _NSTREAM)
            def _(t=t, k=k):
                pltpu.make_async_copy(
                    h_hbm.at[idx_v.at[k]],
                    rows_v.at[b, pl.ds(k * _SL, _SL)], sem_g).wait()
                pltpu.async_copy(rows_v.at[b, pl.ds(k * _SL, _SL)],
                                 out_hbm.at[pl.ds(t * _SL, _SL)], sem_o)

        return carry

    lax.fori_loop(0, _NGRP, group, 0)

    def tail(g, carry):
        b = g % 2
        for k in range(_GG):
            t = (g * _GG + k) * _NW + w

            @pl.when(t < _NSTREAM)
            def _(t=t, k=k):
                pltpu.make_async_copy(
                    rows_v.at[b, pl.ds(k * _SL, _SL)],
                    out_hbm.at[pl.ds(t * _SL, _SL)], sem_o).wait()
        return carry

    lax.fori_loop(_NGRP - 2, _NGRP, tail, 0)


def _sc_gather(h, src):
    return pl.kernel(
        _sc_gather_body,
        out_type=jax.ShapeDtypeStruct((E, W), jnp.float32),
        mesh=_mesh(),
        scratch_types=[
            pltpu.VMEM((_GG, _SL), jnp.int32),
            pltpu.VMEM((2, _GG * _SL, W), jnp.float32),
            pltpu.SemaphoreType.DMA,
            pltpu.SemaphoreType.DMA,
            pltpu.SemaphoreType.DMA,
        ],
        compiler_params=_sc_params,
    )(h, src)


_SGRP = (_HSTREAM + _NS * _GG - 1) // (_NS * _GG)


def _sc_scatter_body(msg_hbm, dst_hbm, zero_hbm, out_hbm, idx_v, rows_v,
                     acc_sh, sem_i, sem_m, sem_a):
    c = lax.axis_index("c")
    s = lax.axis_index("s")
    r0 = s * _ROWS_T
    pltpu.sync_copy(zero_hbm.at[pl.ds(r0, _ROWS_T)], acc_sh.at[pl.ds(r0, _ROWS_T)])
    plsc.subcore_barrier()

    def group(g, carry):
        b = g % 2

        for k in range(_GG):
            t = ((g - 2) * _GG + k) * _NS + s

            @pl.when((g >= 2) & (t < _HSTREAM))
            def _(t=t, k=k):
                pltpu.make_async_copy(
                    rows_v.at[b, pl.ds(k * _SL, _SL)],
                    acc_sh.at[idx_v.at[b, k]], sem_a).wait()

        for k in range(_GG):
            t = (g * _GG + k) * _NS + s

            @pl.when(t < _HSTREAM)
            def _(t=t, k=k):
                off = c * _HALF + t * _SL
                pltpu.async_copy(dst_hbm.at[pl.ds(off, _SL)],
                                 idx_v.at[b, k], sem_i)
                pltpu.async_copy(msg_hbm.at[pl.ds(off, _SL)],
                                 rows_v.at[b, pl.ds(k * _SL, _SL)], sem_m)

        for k in range(_GG):
            t = (g * _GG + k) * _NS + s

            @pl.when(t < _HSTREAM)
            def _(t=t, k=k):
                off = c * _HALF + t * _SL
                pltpu.make_async_copy(dst_hbm.at[pl.ds(off, _SL)],
                                      idx_v.at[b, k], sem_i).wait()
                pltpu.make_async_copy(
                    msg_hbm.at[pl.ds(off, _SL)],
                    rows_v.at[b, pl.ds(k * _SL, _SL)], sem_m).wait()
                pltpu.async_copy(rows_v.at[b, pl.ds(k * _SL, _SL)],
                                 acc_sh.at[idx_v.at[b, k]], sem_a, add=True)

        return carry

    lax.fori_loop(0, _SGRP, group, 0)

    def tail(g, carry):
        b = g % 2
        for k in range(_GG):
            t = (g * _GG + k) * _NS + s

            @pl.when(t < _HSTREAM)
            def _(t=t, k=k):
                pltpu.make_async_copy(
                    rows_v.at[b, pl.ds(k * _SL, _SL)],
                    acc_sh.at[idx_v.at[b, k]], sem_a).wait()
        return carry

    lax.fori_loop(_SGRP - 2, _SGRP, tail, 0)
    plsc.subcore_barrier()
    pltpu.sync_copy(acc_sh.at[pl.ds(r0, _ROWS_T)], out_hbm.at[c, pl.ds(r0, _ROWS_T)])


def _sc_scatter(msg, dst, zeros):
    return pl.kernel(
        _sc_scatter_body,
        out_type=jax.ShapeDtypeStruct((_NC, N, W), jnp.float32),
        mesh=_mesh(),
        scratch_types=[
            pltpu.VMEM((2, _GG, _SL), jnp.int32),
            pltpu.VMEM((2, _GG * _SL, W), jnp.float32),
            pltpu.VMEM_SHARED((N, W), jnp.float32),
            pltpu.SemaphoreType.DMA,
            pltpu.SemaphoreType.DMA,
            pltpu.SemaphoreType.DMA,
        ],
        compiler_params=_sc_params,
    )(msg, dst, zeros)


def _sc_count_body(dst_hbm, zero_hbm, out_hbm, idx_v, ones_v, acc_sh, sem):
    c = lax.axis_index("c")
    s = lax.axis_index("s")

    def fill(i, carry):
        ones_v[i, :] = jnp.ones((16,), jnp.float32)
        return carry

    lax.fori_loop(0, _SL, fill, 0)
    r0 = s * _ROWS_T
    pltpu.sync_copy(zero_hbm.at[pl.ds(r0, _ROWS_T)], acc_sh.at[pl.ds(r0, _ROWS_T)])
    plsc.subcore_barrier()

    def body(i, carry):
        t = i * _NS + s

        @pl.when(t < _HSTREAM)
        def _():
            off = c * _HALF + t * _SL
            pltpu.sync_copy(dst_hbm.at[pl.ds(off, _SL)], idx_v)
            pltpu.sync_copy(ones_v, acc_sh.at[idx_v], add=True)

        return carry

    lax.fori_loop(0, (_HSTREAM + _NS - 1) // _NS, body, 0)
    plsc.subcore_barrier()
    pltpu.sync_copy(acc_sh.at[pl.ds(r0, _ROWS_T)], out_hbm.at[c, pl.ds(r0, _ROWS_T)])


def _sc_count(dst, zeros16):
    return pl.kernel(
        _sc_count_body,
        out_type=jax.ShapeDtypeStruct((_NC, N, 16), jnp.float32),
        mesh=_mesh(),
        scratch_types=[
            pltpu.VMEM((_SL,), jnp.int32),
            pltpu.VMEM((_SL, 16), jnp.float32),
            pltpu.VMEM_SHARED((N, 16), jnp.float32),
            pltpu.SemaphoreType.DMA,
        ],
        compiler_params=_sc_params,
    )(dst, zeros16)


def _layer_norm_c(hc, m, g, b):
    var = jnp.dot(hc * hc, m, preferred_element_type=jnp.float32)
    return hc * lax.rsqrt(var + 1e-5) * g + b


def _node_body(x_ref, w1_ref, b1_ref, w2_ref, b2_ref, m_ref, g_ref, bt_ref,
               out_ref):
    f32 = jnp.float32
    h = jnp.maximum(jnp.dot(x_ref[...], w1_ref[...], preferred_element_type=f32)
                    + b1_ref[...], 0.0)
    h = jnp.dot(h, w2_ref[...], preferred_element_type=f32) + b2_ref[...]
    out_ref[...] = _layer_norm_c(h, m_ref[...], g_ref[...], bt_ref[...])


def _tc_node_encode(x, w1, b1, w2c, b2c, m, g, bt):
    return pl.pallas_call(
        _node_body,
        out_shape=jax.ShapeDtypeStruct((N, W), jnp.float32),
    )(x, w1, b1.reshape(1, W), w2c, b2c.reshape(1, W), m, g.reshape(1, W),
      bt.reshape(1, W))


_B = 640


def _msg_body(ea_ref, hs_ref, eW1_ref, eb1_ref, eW2_ref, eb2_ref, m_ref,
              eg_ref, ebt_ref, kW1_ref, kb1_ref, kW2_ref, kb2_ref, kW3_ref,
              kb3_ref, s_ref, out_ref):
    f32 = jnp.float32
    e = jnp.maximum(jnp.dot(ea_ref[...], eW1_ref[...], preferred_element_type=f32)
                    + eb1_ref[...], 0.0)
    e = jnp.dot(e, eW2_ref[...], preferred_element_type=f32) + eb2_ref[...]
    e = _layer_norm_c(e, m_ref[...], eg_ref[...], ebt_ref[...])
    k1 = jnp.maximum(jnp.dot(e, kW1_ref[...], preferred_element_type=f32)
                     + kb1_ref[...], 0.0)
    k2 = jnp.maximum(jnp.dot(k1.astype(jnp.bfloat16), kW2_ref[...],
                             preferred_element_type=f32)
                     + kb2_ref[...], 0.0)
    wk = jnp.dot(k2.astype(jnp.bfloat16), kW3_ref[...],
                 preferred_element_type=f32) + kb3_ref[...]
    hst = jnp.concatenate([hs_ref[...]] * W, axis=-1)
    prod = wk * hst
    out_ref[...] = jnp.dot(prod, s_ref[...], preferred_element_type=f32)


def _tc_message(edge_attr, hs, eW1, eb1, eW2c, eb2c, m, eg, ebt, kW1l, kb1l,
                kW2l, kb2l, kW3l, kb3l, s):
    def cmap(i):
        return (i, 0)

    def zmap(i):
        return (0, 0)

    return pl.pallas_call(
        _msg_body,
        grid=(E // _B,),
        in_specs=[
            pl.BlockSpec((_B, 4), cmap),
            pl.BlockSpec((_B, W), cmap),
            pl.BlockSpec((4, W), zmap),
            pl.BlockSpec((1, W), zmap),
            pl.BlockSpec((W, W), zmap),
            pl.BlockSpec((1, W), zmap),
            pl.BlockSpec((W, W), zmap),
            pl.BlockSpec((1, W), zmap),
            pl.BlockSpec((1, W), zmap),
            pl.BlockSpec((W, K2), zmap),
            pl.BlockSpec((1, K2), zmap),
            pl.BlockSpec((K2, K2), zmap),
            pl.BlockSpec((1, K2), zmap),
            pl.BlockSpec((K2, W * W), zmap),
            pl.BlockSpec((1, W * W), zmap),
            pl.BlockSpec((W * W, W), zmap),
        ],
        out_specs=pl.BlockSpec((_B, W), cmap),
        out_shape=jax.ShapeDtypeStruct((E, W), jnp.float32),
        compiler_params=pltpu.CompilerParams(
            dimension_semantics=("arbitrary",)),
    )(edge_attr, hs, eW1, eb1.reshape(1, W), eW2c, eb2c.reshape(1, W), m,
      eg.reshape(1, W), ebt.reshape(1, W), kW1l, kb1l.reshape(1, K2), kW2l,
      kb2l.reshape(1, K2), kW3l, kb3l.reshape(1, W * W), s)


def _agg(s_ref, cnt_ref):
    cnt = cnt_ref[0, :, 0:1] + cnt_ref[1, :, 0:1]
    return (s_ref[0, :, :] + s_ref[1, :, :]) / jnp.maximum(cnt, 1.0)


def _update_body(h_ref, s_ref, cnt_ref, out_ref):
    out_ref[...] = jnp.maximum(h_ref[...] + _agg(s_ref, cnt_ref), 0.0)


def _tc_update(h, s, cnt):
    return pl.pallas_call(
        _update_body,
        out_shape=jax.ShapeDtypeStruct((N, W), jnp.float32),
    )(h, s, cnt)


def _update_decode_body(h_ref, s_ref, cnt_ref, w1_ref, b1_ref, w2_ref, b2_ref,
                        out_ref):
    f32 = jnp.float32
    h2 = jnp.maximum(h_ref[...] + _agg(s_ref, cnt_ref), 0.0)
    t = jnp.maximum(jnp.dot(h2, w1_ref[...], preferred_element_type=f32)
                    + b1_ref[...], 0.0)
    out_ref[...] = jnp.dot(t, w2_ref[...], preferred_element_type=f32) + b2_ref[...]


def _tc_update_decode(h, s, cnt, w1, b1, w2, b2):
    return pl.pallas_call(
        _update_decode_body,
        out_shape=jax.ShapeDtypeStruct((N, 1), jnp.float32),
    )(h, s, cnt, w1, b1.reshape(1, W), w2, b2.reshape(1, 1))


def kernel(x, edge_index, edge_attr, nW1, nb1, nW2, nb2, ng, nbt, eW1, eb1,
           eW2, eb2, eg, ebt, kW1, kb1, kW2, kb2, kW3, kb3, dW1, db1, dW2,
           db2):
    src = edge_index[0]
    dst = edge_index[1]
    z32 = jnp.zeros((N, W), jnp.float32)
    z16 = jnp.zeros((N, 16), jnp.float32)
    cen = jnp.eye(W, dtype=jnp.float32) - 1.0 / W
    m = jnp.full((W, W), 1.0 / W, jnp.float32)
    smat = jnp.kron(jnp.eye(W, dtype=jnp.float32), jnp.ones((W, 1), jnp.float32))
    nW2c = nW2 @ cen
    nb2c = nb2 @ cen
    eW2c = eW2 @ cen
    eb2c = eb2 @ cen

    h = _tc_node_encode(x, nW1, nb1, nW2c, nb2c, m, ng, nbt)
    cnt = _sc_count(dst, z16)
    out = None
    for l in range(DEPTH):
        hs = _sc_gather(h, src)
        msg = _tc_message(edge_attr, hs, eW1, eb1, eW2c, eb2c, m, eg, ebt,
                          kW1[l], kb1[l], kW2[l].astype(jnp.bfloat16),
                          kb2[l], kW3[l].astype(jnp.bfloat16), kb3[l], smat)
        s = _sc_scatter(msg, dst, z32)
        if l < DEPTH - 1:
            h = _tc_update(h, s, cnt)
        else:
            out = _tc_update_decode(h, s, cnt, dW1, db1, dW2, db2)
    return out


# speedup vs baseline: 3.8165x; 1.0913x over previous
import functools

import jax
import jax.numpy as jnp
from jax import lax
from jax.experimental import pallas as pl
from jax.experimental.pallas import tpu as pltpu
from jax.experimental.pallas import tpu_sc as plsc

N = 10000
E = 160000
W = 32
K2 = 128
DEPTH = 2

_SL = 128
_NS = 16
_NC = 2
_NW = _NS * _NC
_NSTREAM = E // _SL
_HALF = E // _NC
_HSTREAM = _HALF // _SL
_ROWS_T = N // _NS

_mesh = functools.partial(
    plsc.VectorSubcoreMesh, core_axis_name="c", subcore_axis_name="s")

_sc_params = pltpu.CompilerParams(use_tc_tiling_on_sc=False)


_GG = 8
_NGRP = (_NSTREAM + _NW * _GG - 1) // (_NW * _GG)


def _sc_gather_body(h_hbm, src_hbm, out_hbm, idx_v, rows_v, sem_i, sem_g,
                    sem_o):
    c = lax.axis_index("c")
    s = lax.axis_index("s")
    w = s * _NC + c

    def group(g, carry):
        b = g % 2

        for k in range(_GG):
            t = ((g - 2) * _GG + k) * _NW + w

            @pl.when((g >= 2) & (t < _NSTREAM))
            def _(t=t, k=k):
                pltpu.make_async_copy(
                    rows_v.at[b, pl.ds(k * _SL, _SL)],
                    out_hbm.at[pl.ds(t * _SL, _SL)], sem_o).wait()

        for k in range(_GG):
            t = (g * _GG + k) * _NW + w

            @pl.when(t < _NSTREAM)
            def _(t=t, k=k):
                pltpu.async_copy(src_hbm.at[pl.ds(t * _SL, _SL)],
                                 idx_v.at[k], sem_i)

        for k in range(_GG):
            t = (g * _GG + k) * _NW + w

            @pl.when(t < _NSTREAM)
            def _(t=t, k=k):
                pltpu.make_async_copy(src_hbm.at[pl.ds(t * _SL, _SL)],
                                      idx_v.at[k], sem_i).wait()
                pltpu.async_copy(h_hbm.at[idx_v.at[k]],
                                 rows_v.at[b, pl.ds(k * _SL, _SL)], sem_g)

        for k in range(_GG):
            t = (g * _GG + k) * _NW + w

            @pl.when(t < _NSTREAM)
            def _(t=t, k=k):
                pltpu.make_async_copy(
                    h_hbm.at[idx_v.at[k]],
                    rows_v.at[b, pl.ds(k * _SL, _SL)], sem_g).wait()
                pltpu.async_copy(rows_v.at[b, pl.ds(k * _SL, _SL)],
                                 out_hbm.at[pl.ds(t * _SL, _SL)], sem_o)

        return carry

    lax.fori_loop(0, _NGRP, group, 0)

    def tail(g, carry):
        b = g % 2
        for k in range(_GG):
            t = (g * _GG + k) * _NW + w

            @pl.when(t < _NSTREAM)
            def _(t=t, k=k):
                pltpu.make_async_copy(
                    rows_v.at[b, pl.ds(k * _SL, _SL)],
                    out_hbm.at[pl.ds(t * _SL, _SL)], sem_o).wait()
        return carry

    lax.fori_loop(_NGRP - 2, _NGRP, tail, 0)


def _sc_gather(h, src):
    return pl.kernel(
        _sc_gather_body,
        out_type=jax.ShapeDtypeStruct((E, W), jnp.float32),
        mesh=_mesh(),
        scratch_types=[
            pltpu.VMEM((_GG, _SL), jnp.int32),
            pltpu.VMEM((2, _GG * _SL, W), jnp.float32),
            pltpu.SemaphoreType.DMA,
            pltpu.SemaphoreType.DMA,
            pltpu.SemaphoreType.DMA,
        ],
        compiler_params=_sc_params,
    )(h, src)


_SGRP = (_HSTREAM + _NS * _GG - 1) // (_NS * _GG)


def _sc_scatter_body(msg_hbm, dst_hbm, zero_hbm, out_hbm, idx_v, rows_v,
                     acc_sh, sem_i, sem_m, sem_a):
    c = lax.axis_index("c")
    s = lax.axis_index("s")
    r0 = s * _ROWS_T
    pltpu.sync_copy(zero_hbm.at[pl.ds(r0, _ROWS_T)], acc_sh.at[pl.ds(r0, _ROWS_T)])
    plsc.subcore_barrier()

    def group(g, carry):
        b = g % 2

        for k in range(_GG):
            t = ((g - 2) * _GG + k) * _NS + s

            @pl.when((g >= 2) & (t < _HSTREAM))
            def _(t=t, k=k):
                pltpu.make_async_copy(
                    rows_v.at[b, pl.ds(k * _SL, _SL)],
                    acc_sh.at[idx_v.at[b, k]], sem_a).wait()

        for k in range(_GG):
            t = (g * _GG + k) * _NS + s

            @pl.when(t < _HSTREAM)
            def _(t=t, k=k):
                off = c * _HALF + t * _SL
                pltpu.async_copy(dst_hbm.at[pl.ds(off, _SL)],
                                 idx_v.at[b, k], sem_i)
                pltpu.async_copy(msg_hbm.at[pl.ds(off, _SL)],
                                 rows_v.at[b, pl.ds(k * _SL, _SL)], sem_m)

        for k in range(_GG):
            t = (g * _GG + k) * _NS + s

            @pl.when(t < _HSTREAM)
            def _(t=t, k=k):
                off = c * _HALF + t * _SL
                pltpu.make_async_copy(dst_hbm.at[pl.ds(off, _SL)],
                                      idx_v.at[b, k], sem_i).wait()
                pltpu.make_async_copy(
                    msg_hbm.at[pl.ds(off, _SL)],
                    rows_v.at[b, pl.ds(k * _SL, _SL)], sem_m).wait()
                pltpu.async_copy(rows_v.at[b, pl.ds(k * _SL, _SL)],
                                 acc_sh.at[idx_v.at[b, k]], sem_a, add=True)

        return carry

    lax.fori_loop(0, _SGRP, group, 0)

    def tail(g, carry):
        b = g % 2
        for k in range(_GG):
            t = (g * _GG + k) * _NS + s

            @pl.when(t < _HSTREAM)
            def _(t=t, k=k):
                pltpu.make_async_copy(
                    rows_v.at[b, pl.ds(k * _SL, _SL)],
                    acc_sh.at[idx_v.at[b, k]], sem_a).wait()
        return carry

    lax.fori_loop(_SGRP - 2, _SGRP, tail, 0)
    plsc.subcore_barrier()
    pltpu.sync_copy(acc_sh.at[pl.ds(r0, _ROWS_T)], out_hbm.at[c, pl.ds(r0, _ROWS_T)])


def _sc_scatter(msg, dst, zeros):
    return pl.kernel(
        _sc_scatter_body,
        out_type=jax.ShapeDtypeStruct((_NC, N, W), jnp.float32),
        mesh=_mesh(),
        scratch_types=[
            pltpu.VMEM((2, _GG, _SL), jnp.int32),
            pltpu.VMEM((2, _GG * _SL, W), jnp.float32),
            pltpu.VMEM_SHARED((N, W), jnp.float32),
            pltpu.SemaphoreType.DMA,
            pltpu.SemaphoreType.DMA,
            pltpu.SemaphoreType.DMA,
        ],
        compiler_params=_sc_params,
    )(msg, dst, zeros)


def _sc_count_body(dst_hbm, zero_hbm, out_hbm, idx_v, ones_v, acc_sh, sem):
    c = lax.axis_index("c")
    s = lax.axis_index("s")

    def fill(i, carry):
        ones_v[i, :] = jnp.ones((16,), jnp.float32)
        return carry

    lax.fori_loop(0, _SL, fill, 0)
    r0 = s * _ROWS_T
    pltpu.sync_copy(zero_hbm.at[pl.ds(r0, _ROWS_T)], acc_sh.at[pl.ds(r0, _ROWS_T)])
    plsc.subcore_barrier()

    def body(i, carry):
        t = i * _NS + s

        @pl.when(t < _HSTREAM)
        def _():
            off = c * _HALF + t * _SL
            pltpu.sync_copy(dst_hbm.at[pl.ds(off, _SL)], idx_v)
            pltpu.sync_copy(ones_v, acc_sh.at[idx_v], add=True)

        return carry

    lax.fori_loop(0, (_HSTREAM + _NS - 1) // _NS, body, 0)
    plsc.subcore_barrier()
    pltpu.sync_copy(acc_sh.at[pl.ds(r0, _ROWS_T)], out_hbm.at[c, pl.ds(r0, _ROWS_T)])


def _sc_count(dst, zeros16):
    return pl.kernel(
        _sc_count_body,
        out_type=jax.ShapeDtypeStruct((_NC, N, 16), jnp.float32),
        mesh=_mesh(),
        scratch_types=[
            pltpu.VMEM((_SL,), jnp.int32),
            pltpu.VMEM((_SL, 16), jnp.float32),
            pltpu.VMEM_SHARED((N, 16), jnp.float32),
            pltpu.SemaphoreType.DMA,
        ],
        compiler_params=_sc_params,
    )(dst, zeros16)


def _layer_norm_c(hc, m, g, b):
    var = jnp.dot(hc * hc, m, preferred_element_type=jnp.float32)
    return hc * lax.rsqrt(var + 1e-5) * g + b


def _node_body(x_ref, w1_ref, b1_ref, w2_ref, b2_ref, m_ref, g_ref, bt_ref,
               out_ref):
    f32 = jnp.float32
    h = jnp.maximum(jnp.dot(x_ref[...], w1_ref[...], preferred_element_type=f32)
                    + b1_ref[...], 0.0)
    h = jnp.dot(h, w2_ref[...], preferred_element_type=f32) + b2_ref[...]
    out_ref[...] = _layer_norm_c(h, m_ref[...], g_ref[...], bt_ref[...])


def _tc_node_encode(x, w1, b1, w2c, b2c, m, g, bt):
    return pl.pallas_call(
        _node_body,
        out_shape=jax.ShapeDtypeStruct((N, W), jnp.float32),
    )(x, w1, b1.reshape(1, W), w2c, b2c.reshape(1, W), m, g.reshape(1, W),
      bt.reshape(1, W))


_B = 800


def _msg_body(ea_ref, hs_ref, eW1_ref, eb1_ref, eW2_ref, eb2_ref, m_ref,
              eg_ref, ebt_ref, kW1_ref, kb1_ref, kW2_ref, kb2_ref, kW3_ref,
              kb3_ref, s_ref, out_ref):
    f32 = jnp.float32
    e = jnp.maximum(jnp.dot(ea_ref[...], eW1_ref[...], preferred_element_type=f32)
                    + eb1_ref[...], 0.0)
    e = jnp.dot(e, eW2_ref[...], preferred_element_type=f32) + eb2_ref[...]
    e = _layer_norm_c(e, m_ref[...], eg_ref[...], ebt_ref[...])
    k1 = jnp.maximum(jnp.dot(e, kW1_ref[...], preferred_element_type=f32)
                     + kb1_ref[...], 0.0)
    k2 = jnp.maximum(jnp.dot(k1.astype(jnp.bfloat16), kW2_ref[...],
                             preferred_element_type=f32)
                     + kb2_ref[...], 0.0)
    wk = jnp.dot(k2.astype(jnp.bfloat16), kW3_ref[...],
                 preferred_element_type=f32).astype(jnp.bfloat16) + kb3_ref[...]
    hst = jnp.concatenate([hs_ref[...].astype(jnp.bfloat16)] * W, axis=-1)
    prod = wk * hst
    out_ref[...] = jnp.dot(prod, s_ref[...], preferred_element_type=f32)


def _tc_message(edge_attr, hs, eW1, eb1, eW2c, eb2c, m, eg, ebt, kW1l, kb1l,
                kW2l, kb2l, kW3l, kb3l, s):
    def cmap(i):
        return (i, 0)

    def zmap(i):
        return (0, 0)

    return pl.pallas_call(
        _msg_body,
        grid=(E // _B,),
        in_specs=[
            pl.BlockSpec((_B, 4), cmap),
            pl.BlockSpec((_B, W), cmap),
            pl.BlockSpec((4, W), zmap),
            pl.BlockSpec((1, W), zmap),
            pl.BlockSpec((W, W), zmap),
            pl.BlockSpec((1, W), zmap),
            pl.BlockSpec((W, W), zmap),
            pl.BlockSpec((1, W), zmap),
            pl.BlockSpec((1, W), zmap),
            pl.BlockSpec((W, K2), zmap),
            pl.BlockSpec((1, K2), zmap),
            pl.BlockSpec((K2, K2), zmap),
            pl.BlockSpec((1, K2), zmap),
            pl.BlockSpec((K2, W * W), zmap),
            pl.BlockSpec((1, W * W), zmap),
            pl.BlockSpec((W * W, W), zmap),
        ],
        out_specs=pl.BlockSpec((_B, W), cmap),
        out_shape=jax.ShapeDtypeStruct((E, W), jnp.float32),
        compiler_params=pltpu.CompilerParams(
            dimension_semantics=("arbitrary",)),
    )(edge_attr, hs, eW1, eb1.reshape(1, W), eW2c, eb2c.reshape(1, W), m,
      eg.reshape(1, W), ebt.reshape(1, W), kW1l, kb1l.reshape(1, K2), kW2l,
      kb2l.reshape(1, K2), kW3l, kb3l.reshape(1, W * W), s)


def _agg(s_ref, cnt_ref):
    cnt = cnt_ref[0, :, 0:1] + cnt_ref[1, :, 0:1]
    return (s_ref[0, :, :] + s_ref[1, :, :]) / jnp.maximum(cnt, 1.0)


def _update_body(h_ref, s_ref, cnt_ref, out_ref):
    out_ref[...] = jnp.maximum(h_ref[...] + _agg(s_ref, cnt_ref), 0.0)


def _tc_update(h, s, cnt):
    return pl.pallas_call(
        _update_body,
        out_shape=jax.ShapeDtypeStruct((N, W), jnp.float32),
    )(h, s, cnt)


def _update_decode_body(h_ref, s_ref, cnt_ref, w1_ref, b1_ref, w2_ref, b2_ref,
                        out_ref):
    f32 = jnp.float32
    h2 = jnp.maximum(h_ref[...] + _agg(s_ref, cnt_ref), 0.0)
    t = jnp.maximum(jnp.dot(h2, w1_ref[...], preferred_element_type=f32)
                    + b1_ref[...], 0.0)
    out_ref[...] = jnp.dot(t, w2_ref[...], preferred_element_type=f32) + b2_ref[...]


def _tc_update_decode(h, s, cnt, w1, b1, w2, b2):
    return pl.pallas_call(
        _update_decode_body,
        out_shape=jax.ShapeDtypeStruct((N, 1), jnp.float32),
    )(h, s, cnt, w1, b1.reshape(1, W), w2, b2.reshape(1, 1))


def kernel(x, edge_index, edge_attr, nW1, nb1, nW2, nb2, ng, nbt, eW1, eb1,
           eW2, eb2, eg, ebt, kW1, kb1, kW2, kb2, kW3, kb3, dW1, db1, dW2,
           db2):
    src = edge_index[0]
    dst = edge_index[1]
    z32 = jnp.zeros((N, W), jnp.float32)
    z16 = jnp.zeros((N, 16), jnp.float32)
    cen = jnp.eye(W, dtype=jnp.float32) - 1.0 / W
    m = jnp.full((W, W), 1.0 / W, jnp.float32)
    smat = jnp.kron(jnp.eye(W, dtype=jnp.float32), jnp.ones((W, 1), jnp.float32))
    nW2c = nW2 @ cen
    nb2c = nb2 @ cen
    eW2c = eW2 @ cen
    eb2c = eb2 @ cen

    h = _tc_node_encode(x, nW1, nb1, nW2c, nb2c, m, ng, nbt)
    cnt = _sc_count(dst, z16)
    out = None
    for l in range(DEPTH):
        hs = _sc_gather(h, src)
        msg = _tc_message(edge_attr, hs, eW1, eb1, eW2c, eb2c, m, eg, ebt,
                          kW1[l], kb1[l], kW2[l].astype(jnp.bfloat16),
                          kb2[l], kW3[l].astype(jnp.bfloat16),
                          kb3[l].astype(jnp.bfloat16),
                          smat.astype(jnp.bfloat16))
        s = _sc_scatter(msg, dst, z32)
        if l < DEPTH - 1:
            h = _tc_update(h, s, cnt)
        else:
            out = _tc_update_decode(h, s, cnt, dW1, db1, dW2, db2)
    return out
